# Initial kernel scaffold; baseline (speedup 1.0000x reference)
import functools

import jax
import jax.numpy as jnp
from jax import lax
from jax.experimental import pallas as pl
from jax.experimental.pallas import tpu as pltpu
from jax.experimental.pallas import tpu_sc as plsc

N_NODES = 10000
N_EDGES = 320000
D = 128
DE = 16

NC = 2
NS = 16
NW = NC * NS
E_PER_W = N_EDGES // NW
CHUNK = 80
N_CHUNKS = E_PER_W // CHUNK
ROWS_PER_TILE = N_NODES // NS


_sc_mesh = plsc.VectorSubcoreMesh(core_axis_name="c", subcore_axis_name="s")


@functools.partial(
    pl.kernel,
    mesh=_sc_mesh,
    out_type=(
        jax.ShapeDtypeStruct((NC, N_NODES, D), jnp.float32),
        jax.ShapeDtypeStruct((NC, N_NODES, DE), jnp.float32),
        jax.ShapeDtypeStruct((NC, N_NODES, DE), jnp.float32),
    ),
    scratch_types=[
        pltpu.VMEM((CHUNK,), jnp.int32),
        pltpu.VMEM((CHUNK,), jnp.int32),
        pltpu.VMEM((CHUNK, D), jnp.float32),
        pltpu.VMEM((CHUNK, DE), jnp.float32),
        pltpu.VMEM((CHUNK, DE), jnp.float32),
        pltpu.VMEM_SHARED((N_NODES, D), jnp.float32),
        pltpu.VMEM_SHARED((N_NODES, DE), jnp.float32),
        pltpu.VMEM_SHARED((N_NODES, DE), jnp.float32),
        pltpu.SemaphoreType.DMA,
    ],
)
def _sc_aggregate(x_hbm, src_hbm, dst_hbm, ea_hbm, z_d_hbm, z_de_hbm,
                  out_s, out_e, out_d,
                  src_v, dst_v, rows_v, ea_v, ones_v,
                  acc_s, acc_e, acc_d, sem):
    c = lax.axis_index("c")
    s = lax.axis_index("s")
    wid = s * NC + c

    r0 = s * ROWS_PER_TILE
    pltpu.sync_copy(z_d_hbm.at[pl.ds(r0, ROWS_PER_TILE)],
                    acc_s.at[pl.ds(r0, ROWS_PER_TILE)])
    pltpu.sync_copy(z_de_hbm.at[pl.ds(r0, ROWS_PER_TILE)],
                    acc_e.at[pl.ds(r0, ROWS_PER_TILE)])
    pltpu.sync_copy(z_de_hbm.at[pl.ds(r0, ROWS_PER_TILE)],
                    acc_d.at[pl.ds(r0, ROWS_PER_TILE)])

    onehot = jnp.where(lax.broadcasted_iota(jnp.int32, (DE,), 0) == 0,
                       1.0, 0.0).astype(jnp.float32)

    def init_ones(i, carry):
        ones_v[i, :] = onehot
        return carry

    lax.fori_loop(0, CHUNK, init_ones, 0)

    plsc.subcore_barrier()

    ebase = wid * E_PER_W

    def body(i, carry):
        base = ebase + i * CHUNK
        pltpu.sync_copy(src_hbm.at[pl.ds(base, CHUNK)], src_v)
        pltpu.sync_copy(dst_hbm.at[pl.ds(base, CHUNK)], dst_v)
        pltpu.sync_copy(ea_hbm.at[pl.ds(base, CHUNK)], ea_v)
        pltpu.async_copy(x_hbm.at[src_v], rows_v, sem).wait()
        pltpu.sync_copy(rows_v, acc_s.at[dst_v], add=True)
        pltpu.sync_copy(ea_v, acc_e.at[dst_v], add=True)
        pltpu.sync_copy(ones_v, acc_d.at[dst_v], add=True)
        return carry

    lax.fori_loop(0, N_CHUNKS, body, 0)

    plsc.subcore_barrier()

    pltpu.sync_copy(acc_s.at[pl.ds(r0, ROWS_PER_TILE)],
                    out_s.at[c, pl.ds(r0, ROWS_PER_TILE)])
    pltpu.sync_copy(acc_e.at[pl.ds(r0, ROWS_PER_TILE)],
                    out_e.at[c, pl.ds(r0, ROWS_PER_TILE)])
    pltpu.sync_copy(acc_d.at[pl.ds(r0, ROWS_PER_TILE)],
                    out_d.at[c, pl.ds(r0, ROWS_PER_TILE)])


BLK = 1000


def _dense_body(x_ref, s_ref, e_ref, d_ref,
                w1a_ref, w1b_ref, w1c_ref, b1_ref,
                w2a_ref, w2b_ref, b2_ref, o_ref):
    x = x_ref[...]
    s = s_ref[0] + s_ref[1]
    e = e_ref[0] + e_ref[1]
    deg = d_ref[0, :, 0:1] + d_ref[1, :, 0:1]
    agg = jnp.dot(s, w1a_ref[...], preferred_element_type=jnp.float32)
    agg += deg * (jnp.dot(x, w1b_ref[...], preferred_element_type=jnp.float32)
                  + b1_ref[...])
    agg += jnp.dot(e, w1c_ref[...], preferred_element_type=jnp.float32)
    o = jnp.dot(agg, w2a_ref[...], preferred_element_type=jnp.float32)
    o += jnp.dot(x, w2b_ref[...], preferred_element_type=jnp.float32)
    o += b2_ref[...]
    o_ref[...] = o


def _dense(x, s_part, e_part, d_part, w1a, w1b, w1c, b1, w2a, w2b, b2):
    grid = (N_NODES // BLK,)

    def full(shape):
        return pl.BlockSpec(shape, lambda i: tuple(0 for _ in shape))

    return pl.pallas_call(
        _dense_body,
        grid=grid,
        in_specs=[
            pl.BlockSpec((BLK, D), lambda i: (i, 0)),
            pl.BlockSpec((NC, BLK, D), lambda i: (0, i, 0)),
            pl.BlockSpec((NC, BLK, DE), lambda i: (0, i, 0)),
            pl.BlockSpec((NC, BLK, DE), lambda i: (0, i, 0)),
            full((D, D)),
            full((D, D)),
            full((DE, D)),
            full((1, D)),
            full((D, D)),
            full((D, D)),
            full((1, D)),
        ],
        out_specs=pl.BlockSpec((BLK, D), lambda i: (i, 0)),
        out_shape=jax.ShapeDtypeStruct((N_NODES, D), jnp.float32),
    )(x, s_part, e_part, d_part, w1a, w1b, w1c, b1, w2a, w2b, b2)


def kernel(x, edge_index, edge_attr, W1, b1, W2, b2):
    src = edge_index[0]
    dst = edge_index[1]
    w1a, w1b, w1c = W1[:D], W1[D:2 * D], W1[2 * D:]
    w2a, w2b = W2[:D], W2[D:]
    z_d = jnp.zeros((N_NODES, D), jnp.float32)
    z_de = jnp.zeros((N_NODES, DE), jnp.float32)
    s_part, e_part, d_part = _sc_aggregate(x, src, dst, edge_attr, z_d, z_de)
    return _dense(x, s_part, e_part, d_part, w1a, w1b, w1c,
                  b1.reshape(1, D), w2a, w2b, b2.reshape(1, D))


# speedup vs baseline: 3.3518x; 3.3518x over previous
import functools

import jax
import jax.numpy as jnp
from jax import lax
from jax.experimental import pallas as pl
from jax.experimental.pallas import tpu as pltpu
from jax.experimental.pallas import tpu_sc as plsc

N_NODES = 10000
N_EDGES = 320000
D = 128
DE = 16
DE2 = 32

NC = 2
NS = 16
NW = NC * NS
E_PER_W = N_EDGES // NW
CHUNK = 40
N_CHUNKS = E_PER_W // CHUNK
RPT = 640
ZCH = 64
N_PAD = N_NODES


_sc_mesh = plsc.VectorSubcoreMesh(core_axis_name="c", subcore_axis_name="s")


@functools.partial(
    pl.kernel,
    mesh=_sc_mesh,
    compiler_params=pltpu.CompilerParams(use_tc_tiling_on_sc=False),
    out_type=(
        jax.ShapeDtypeStruct((NC * N_PAD, D), jnp.float32),
        jax.ShapeDtypeStruct((NC * N_PAD, DE2), jnp.float32),
    ),
    scratch_types=[
        pltpu.VMEM((CHUNK,), jnp.int32),
        pltpu.VMEM((CHUNK,), jnp.int32),
        pltpu.VMEM((CHUNK, D), jnp.float32),
        pltpu.VMEM((CHUNK, DE2), jnp.float32),
        pltpu.VMEM((ZCH, D), jnp.float32),
        pltpu.VMEM((ZCH, DE2), jnp.float32),
        pltpu.VMEM_SHARED((N_PAD, D), jnp.float32),
        pltpu.VMEM_SHARED((N_PAD, DE2), jnp.float32),
        pltpu.SemaphoreType.DMA,
    ],
)
def _sc_aggregate(x_hbm, src_hbm, dst_hbm, ea_hbm, z_d_hbm, z_de_hbm,
                  out_s, out_e,
                  src_v, dst_v, rows_v, ea_v, ch_d, ch_de,
                  acc_s, acc_e, sem):
    c = lax.axis_index("c")
    s = lax.axis_index("s")
    wid = s * NC + c

    r0 = jnp.minimum(s * RPT, N_PAD - RPT)
    pltpu.sync_copy(z_d_hbm, ch_d)
    pltpu.sync_copy(z_de_hbm, ch_de)

    @pl.loop(0, RPT // ZCH)
    def zbody(j):
        pltpu.async_copy(ch_d, acc_s.at[pl.ds(r0 + j * ZCH, ZCH)], sem).wait()
        pltpu.async_copy(ch_de, acc_e.at[pl.ds(r0 + j * ZCH, ZCH)], sem).wait()

    plsc.subcore_barrier()

    ebase = wid * E_PER_W

    @pl.loop(0, N_CHUNKS)
    def body(i):
        base = ebase + i * CHUNK
        pltpu.async_copy(src_hbm.at[pl.ds(base, CHUNK)], src_v, sem).wait()
        pltpu.async_copy(dst_hbm.at[pl.ds(base, CHUNK)], dst_v, sem).wait()
        pltpu.async_copy(ea_hbm.at[pl.ds(base, CHUNK)], ea_v, sem).wait()
        pltpu.async_copy(x_hbm.at[src_v], rows_v, sem).wait()
        pltpu.async_copy(rows_v, acc_s.at[dst_v], sem, add=True).wait()
        pltpu.async_copy(ea_v, acc_e.at[dst_v], sem, add=True).wait()

    plsc.subcore_barrier()

    obase = c * N_PAD + r0

    @pl.loop(0, RPT // ZCH)
    def obody(j):
        pltpu.async_copy(acc_s.at[pl.ds(r0 + j * ZCH, ZCH)], ch_d, sem).wait()
        pltpu.async_copy(ch_d, out_s.at[pl.ds(obase + j * ZCH, ZCH)], sem).wait()
        pltpu.async_copy(acc_e.at[pl.ds(r0 + j * ZCH, ZCH)], ch_de, sem).wait()
        pltpu.async_copy(ch_de, out_e.at[pl.ds(obase + j * ZCH, ZCH)], sem).wait()


BLK = 1000


def _dense_body(x_ref, s_ref, e_ref,
                w1a_ref, w1b_ref, w1c_ref, b1_ref,
                w2a_ref, w2b_ref, b2_ref, o_ref):
    x = x_ref[...]
    s = s_ref[0] + s_ref[1]
    e = e_ref[0] + e_ref[1]
    deg = e[:, DE:DE + 1]
    agg = jnp.dot(s, w1a_ref[...], preferred_element_type=jnp.float32)
    agg += deg * (jnp.dot(x, w1b_ref[...], preferred_element_type=jnp.float32)
                  + b1_ref[...])
    agg += jnp.dot(e, w1c_ref[...], preferred_element_type=jnp.float32)
    o = jnp.dot(agg, w2a_ref[...], preferred_element_type=jnp.float32)
    o += jnp.dot(x, w2b_ref[...], preferred_element_type=jnp.float32)
    o += b2_ref[...]
    o_ref[...] = o


def _dense(x, s_part, e_part, w1a, w1b, w1c, b1, w2a, w2b, b2):
    grid = (N_NODES // BLK,)

    def full(shape):
        return pl.BlockSpec(shape, lambda i: tuple(0 for _ in shape))

    return pl.pallas_call(
        _dense_body,
        grid=grid,
        in_specs=[
            pl.BlockSpec((BLK, D), lambda i: (i, 0)),
            pl.BlockSpec((NC, BLK, D), lambda i: (0, i, 0)),
            pl.BlockSpec((NC, BLK, DE2), lambda i: (0, i, 0)),
            full((D, D)),
            full((D, D)),
            full((DE2, D)),
            full((1, D)),
            full((D, D)),
            full((D, D)),
            full((1, D)),
        ],
        out_specs=pl.BlockSpec((BLK, D), lambda i: (i, 0)),
        out_shape=jax.ShapeDtypeStruct((N_NODES, D), jnp.float32),
    )(x, s_part, e_part, w1a, w1b, w1c, b1, w2a, w2b, b2)


def kernel(x, edge_index, edge_attr, W1, b1, W2, b2):
    src = edge_index[0]
    dst = edge_index[1]
    w1a, w1b, w1c = W1[:D], W1[D:2 * D], W1[2 * D:]
    w1c_pad = jnp.zeros((DE2, D), jnp.float32).at[:DE].set(w1c)
    w2a, w2b = W2[:D], W2[D:]
    ea_ext = jnp.concatenate(
        [edge_attr,
         jnp.ones((N_EDGES, 1), jnp.float32),
         jnp.zeros((N_EDGES, DE2 - DE - 1), jnp.float32)], axis=1)
    z_d = jnp.zeros((ZCH, D), jnp.float32)
    z_de = jnp.zeros((ZCH, DE2), jnp.float32)
    s_part, e_part = _sc_aggregate(x, src, dst, ea_ext, z_d, z_de)
    s_part = s_part.reshape(NC, N_PAD, D)
    e_part = e_part.reshape(NC, N_PAD, DE2)
    return _dense(x, s_part, e_part, w1a, w1b, w1c_pad,
                  b1.reshape(1, D), w2a, w2b, b2.reshape(1, D))


# speedup vs baseline: 6.2341x; 1.8599x over previous
import functools

import jax
import jax.numpy as jnp
from jax import lax
from jax.experimental import pallas as pl
from jax.experimental.pallas import tpu as pltpu
from jax.experimental.pallas import tpu_sc as plsc

N_NODES = 10000
N_EDGES = 320000
D = 128
DE = 16

NC = 2
NS = 16
NW = NC * NS
E_PER_W = N_EDGES // NW
CHUNK = 80
N_CHUNKS = E_PER_W // CHUNK
RPT = 640
N_PAD = N_NODES


_sc_mesh = plsc.VectorSubcoreMesh(core_axis_name="c", subcore_axis_name="s")


@functools.partial(
    pl.kernel,
    mesh=_sc_mesh,
    compiler_params=pltpu.CompilerParams(use_tc_tiling_on_sc=False),
    out_type=(
        jax.ShapeDtypeStruct((NC * N_PAD, D), jnp.float32),
        jax.ShapeDtypeStruct((NC * N_PAD, DE), jnp.float32),
        jax.ShapeDtypeStruct((NC * N_PAD, DE), jnp.float32),
    ),
    scratch_types=[
        pltpu.VMEM((CHUNK,), jnp.int32),
        pltpu.VMEM((CHUNK,), jnp.int32),
        pltpu.VMEM((CHUNK, D), jnp.float32),
        pltpu.VMEM((CHUNK, DE), jnp.float32),
        pltpu.VMEM((CHUNK, DE), jnp.float32),
        pltpu.VMEM_SHARED((N_PAD, D), jnp.float32),
        pltpu.VMEM_SHARED((N_PAD, DE), jnp.float32),
        pltpu.VMEM_SHARED((N_PAD, DE), jnp.float32),
        pltpu.SemaphoreType.DMA,
        pltpu.SemaphoreType.DMA,
        pltpu.SemaphoreType.DMA,
    ],
)
def _sc_aggregate(x_hbm, src_hbm, dst_hbm, ea_hbm, onehot_hbm,
                  z_d_hbm, z_de_hbm,
                  out_s, out_e, out_d,
                  src_v, dst_v, rows_v, ea_v, ones_v,
                  acc_s, acc_e, acc_d, sem, sem2, sem3):
    c = lax.axis_index("c")
    s = lax.axis_index("s")
    wid = s * NC + c

    pltpu.sync_copy(onehot_hbm, ones_v)
    pltpu.sync_copy(z_d_hbm, rows_v)
    pltpu.sync_copy(z_de_hbm, ea_v)

    r0 = jnp.minimum(s * RPT, N_PAD - RPT)

    @pl.loop(0, RPT // CHUNK)
    def zbody(j):
        pltpu.async_copy(rows_v, acc_s.at[pl.ds(r0 + j * CHUNK, CHUNK)],
                         sem).wait()
        pltpu.async_copy(ea_v, acc_e.at[pl.ds(r0 + j * CHUNK, CHUNK)],
                         sem).wait()
        pltpu.async_copy(ea_v, acc_d.at[pl.ds(r0 + j * CHUNK, CHUNK)],
                         sem).wait()

    plsc.subcore_barrier()

    ebase = wid * E_PER_W

    @pl.loop(0, N_CHUNKS)
    def body(i):
        base = ebase + i * CHUNK
        a = pltpu.async_copy(src_hbm.at[pl.ds(base, CHUNK)], src_v, sem)
        b = pltpu.async_copy(dst_hbm.at[pl.ds(base, CHUNK)], dst_v, sem2)
        e = pltpu.async_copy(ea_hbm.at[pl.ds(base, CHUNK)], ea_v, sem3)
        a.wait()
        b.wait()
        e.wait()
        pltpu.async_copy(x_hbm.at[src_v], rows_v, sem).wait()
        sa = pltpu.async_copy(rows_v, acc_s.at[dst_v], sem, add=True)
        sb = pltpu.async_copy(ea_v, acc_e.at[dst_v], sem2, add=True)
        sc = pltpu.async_copy(ones_v, acc_d.at[dst_v], sem3, add=True)
        sa.wait()
        sb.wait()
        sc.wait()

    plsc.subcore_barrier()

    obase = c * N_PAD + r0

    @pl.loop(0, RPT // CHUNK)
    def obody(j):
        pltpu.async_copy(acc_s.at[pl.ds(r0 + j * CHUNK, CHUNK)], rows_v,
                         sem).wait()
        pltpu.async_copy(rows_v, out_s.at[pl.ds(obase + j * CHUNK, CHUNK)],
                         sem).wait()
        pltpu.async_copy(acc_e.at[pl.ds(r0 + j * CHUNK, CHUNK)], ea_v,
                         sem).wait()
        pltpu.async_copy(ea_v, out_e.at[pl.ds(obase + j * CHUNK, CHUNK)],
                         sem).wait()
        pltpu.async_copy(acc_d.at[pl.ds(r0 + j * CHUNK, CHUNK)], ea_v,
                         sem).wait()
        pltpu.async_copy(ea_v, out_d.at[pl.ds(obase + j * CHUNK, CHUNK)],
                         sem).wait()


BLK = 1000


def _dense_body(x_ref, s_ref, e_ref, d_ref,
                w1a_ref, w1b_ref, w1c_ref, b1_ref,
                w2a_ref, w2b_ref, b2_ref, o_ref):
    x = x_ref[...]
    s = s_ref[0] + s_ref[1]
    e = e_ref[0] + e_ref[1]
    deg = d_ref[0, :, 0:1] + d_ref[1, :, 0:1]
    agg = jnp.dot(s, w1a_ref[...], preferred_element_type=jnp.float32)
    agg += deg * (jnp.dot(x, w1b_ref[...], preferred_element_type=jnp.float32)
                  + b1_ref[...])
    agg += jnp.dot(e, w1c_ref[...], preferred_element_type=jnp.float32)
    o = jnp.dot(agg, w2a_ref[...], preferred_element_type=jnp.float32)
    o += jnp.dot(x, w2b_ref[...], preferred_element_type=jnp.float32)
    o += b2_ref[...]
    o_ref[...] = o


def _dense(x, s_part, e_part, d_part, w1a, w1b, w1c, b1, w2a, w2b, b2):
    grid = (N_NODES // BLK,)

    def full(shape):
        return pl.BlockSpec(shape, lambda i: tuple(0 for _ in shape))

    return pl.pallas_call(
        _dense_body,
        grid=grid,
        in_specs=[
            pl.BlockSpec((BLK, D), lambda i: (i, 0)),
            pl.BlockSpec((NC, BLK, D), lambda i: (0, i, 0)),
            pl.BlockSpec((NC, BLK, DE), lambda i: (0, i, 0)),
            pl.BlockSpec((NC, BLK, DE), lambda i: (0, i, 0)),
            full((D, D)),
            full((D, D)),
            full((DE, D)),
            full((1, D)),
            full((D, D)),
            full((D, D)),
            full((1, D)),
        ],
        out_specs=pl.BlockSpec((BLK, D), lambda i: (i, 0)),
        out_shape=jax.ShapeDtypeStruct((N_NODES, D), jnp.float32),
    )(x, s_part, e_part, d_part, w1a, w1b, w1c, b1, w2a, w2b, b2)


def kernel(x, edge_index, edge_attr, W1, b1, W2, b2):
    src = edge_index[0]
    dst = edge_index[1]
    w1a, w1b, w1c = W1[:D], W1[D:2 * D], W1[2 * D:]
    w2a, w2b = W2[:D], W2[D:]
    onehot = jnp.zeros((CHUNK, DE), jnp.float32).at[:, 0].set(1.0)
    z_d = jnp.zeros((CHUNK, D), jnp.float32)
    z_de = jnp.zeros((CHUNK, DE), jnp.float32)
    s_part, e_part, d_part = _sc_aggregate(x, src, dst, edge_attr, onehot,
                                           z_d, z_de)
    s_part = s_part.reshape(NC, N_PAD, D)
    e_part = e_part.reshape(NC, N_PAD, DE)
    d_part = d_part.reshape(NC, N_PAD, DE)
    return _dense(x, s_part, e_part, d_part, w1a, w1b, w1c,
                  b1.reshape(1, D), w2a, w2b, b2.reshape(1, D))
